# Initial kernel scaffold; baseline (speedup 1.0000x reference)
import jax
import jax.numpy as jnp
from jax.experimental import pallas as pl


def kernel(x, edge_index, W, att_src, att_dst, bias):
    raise NotImplementedError("write your pallas kernel here")


# speedup vs baseline: 28.8504x; 28.8504x over previous
import functools

import jax
import jax.numpy as jnp
from jax import lax
from jax.experimental import pallas as pl
from jax.experimental.pallas import tpu as pltpu
from jax.experimental.pallas import tpu_sc as plsc

N = 10000
E = 320000
IN = 128
H = 8
C = 16
HC = H * C

E1 = E + N
NC, NS = 2, 16
NW = NC * NS

EP = 360448
T1 = EP // NS
T2 = EP // NW
IR1 = T1 // 128
IR2 = T2 // 128
RB = 8
RQ = 2 * RB
SUB = 128
SUB2 = 64
NZ = 624

_DNUMS = lax.GatherDimensionNumbers(
    offset_dims=(), collapsed_slice_dims=(0,), start_index_map=(0,))


def _dyngather(x, idx):
    return lax.gather(x, idx[:, None], _DNUMS, slice_sizes=(1,),
                      mode=lax.GatherScatterMode.PROMISE_IN_BOUNDS)


def _tc_prep(x_ref, w_ref, mcat_ref, p_ref, xw_ref, asd_ref, srow_ref):
    xw = jnp.dot(x_ref[...], w_ref[...], preferred_element_type=jnp.float32)
    xw_ref[...] = xw
    asd = jnp.dot(xw, mcat_ref[...], preferred_element_type=jnp.float32)
    asd_ref[...] = asd
    m = jnp.max(asd, axis=0, keepdims=True)
    s = jnp.dot(m, p_ref[...], preferred_element_type=jnp.float32)
    srow_ref[...] = jnp.maximum(s, 0.2 * s)


def _tc_final(acc_ref, x_ref, b_ref, out_ref):
    out_ref[...] = acc_ref[0] + acc_ref[1] + x_ref[...] + b_ref[...]


def _sc_body(asd, xw, srcq, dstq, shift, z128, z16,
             expalpha, outacc,
             sh_out, sh_den, sh_asd, srcv, dstv, svbuf, dvbuf,
             xwbuf, shiftv):
    c = lax.axis_index("c")
    s = lax.axis_index("s")
    w = s * NC + c

    lane = lax.iota(jnp.int32, 16)
    hi_idx = (lane & 7) + 8
    lo_maskf = jnp.clip(8.0 - lane.astype(jnp.float32), 0.0, 1.0)

    pltpu.sync_copy(z128.at[pl.ds(s * NZ, NZ)], sh_out.at[pl.ds(s * NZ, NZ)])
    pltpu.sync_copy(z16.at[pl.ds(s * NZ, NZ)], sh_den.at[pl.ds(s * NZ, NZ)])
    pltpu.sync_copy(asd.at[pl.ds(s * NZ, NZ)], sh_asd.at[pl.ds(s * NZ, NZ)])

    @pl.when(s == 0)
    def _():
        pltpu.sync_copy(z128.at[pl.ds(NS * NZ, N - NS * NZ)],
                        sh_out.at[pl.ds(NS * NZ, N - NS * NZ)])
        pltpu.sync_copy(z16.at[pl.ds(NS * NZ, N - NS * NZ)],
                        sh_den.at[pl.ds(NS * NZ, N - NS * NZ)])
        pltpu.sync_copy(asd.at[pl.ds(NS * NZ, N - NS * NZ)],
                        sh_asd.at[pl.ds(NS * NZ, N - NS * NZ)])

    pltpu.sync_copy(shift.at[pl.ds(0, 16)], shiftv)
    plsc.subcore_barrier()

    sv = shiftv[...]

    def p1_blk(ib, _):
        e0 = (s * IR1 + ib * RB) * 128
        rq0 = (s * IR1 + ib * RB) * 2

        @pl.when(e0 < E1)
        def _():
            pltpu.sync_copy(srcq.at[pl.ds(rq0, RQ)], srcv)
            pltpu.sync_copy(dstq.at[pl.ds(rq0, RQ)], dstv)

            def p1_sub(j, _):
                es = e0 + j * SUB

                @pl.when(es < E1)
                def _():
                    pltpu.sync_copy(sh_asd.at[srcv.at[2 * j]],
                                    svbuf.at[pl.ds(0, 64)])
                    pltpu.sync_copy(sh_asd.at[srcv.at[2 * j + 1]],
                                    svbuf.at[pl.ds(64, 64)])
                    pltpu.sync_copy(sh_asd.at[dstv.at[2 * j]],
                                    dvbuf.at[pl.ds(0, 64)])
                    pltpu.sync_copy(sh_asd.at[dstv.at[2 * j + 1]],
                                    dvbuf.at[pl.ds(64, 64)])

                    def edge(e, _):
                        srow = svbuf[e, :]
                        drow = _dyngather(dvbuf[e, :], hi_idx)
                        t = srow + drow
                        t = jnp.maximum(t, 0.2 * t)
                        z = jnp.exp(t - sv)
                        padf = jnp.clip((E1 - (es + e)).astype(jnp.float32),
                                        0.0, 1.0)
                        svbuf[e, :] = z * (lo_maskf * padf)
                        return 0
                    lax.fori_loop(0, SUB, edge, 0)

                    pltpu.sync_copy(svbuf, expalpha.at[pl.ds(es, SUB)])
                    pltpu.sync_copy(svbuf.at[pl.ds(0, 64)],
                                    sh_den.at[dstv.at[2 * j]], add=True)
                    pltpu.sync_copy(svbuf.at[pl.ds(64, 64)],
                                    sh_den.at[dstv.at[2 * j + 1]], add=True)
                return 0
            lax.fori_loop(0, RB, p1_sub, 0)
        return 0
    lax.fori_loop(0, IR1 // RB, p1_blk, 0)
    plsc.subcore_barrier()

    ev, dbuf = svbuf, dvbuf

    def p2_blk(ib, _):
        e0 = (w * IR2 + ib * RB) * 128
        rq0 = (w * IR2 + ib * RB) * 2

        @pl.when(e0 < E1)
        def _():
            pltpu.sync_copy(srcq.at[pl.ds(rq0, RQ)], srcv)
            pltpu.sync_copy(dstq.at[pl.ds(rq0, RQ)], dstv)

            def p2_sub(j, _):
                es = e0 + j * SUB2

                @pl.when(es < E1)
                def _():
                    pltpu.sync_copy(expalpha.at[pl.ds(es, SUB2)],
                                    ev.at[pl.ds(0, SUB2)])
                    pltpu.sync_copy(xw.at[srcv.at[j]], xwbuf)
                    pltpu.sync_copy(sh_den.at[dstv.at[j]],
                                    dbuf.at[pl.ds(0, SUB2)])

                    def edge(e, _):
                        a = ev[e, :] / (dbuf[e, :] + 1e-16)
                        ev[e, :] = a
                        for h in range(H):
                            sp = _dyngather(a, jnp.full((16,), h, jnp.int32))
                            xwbuf[e, pl.ds(h * 16, 16)] = (
                                xwbuf[e, pl.ds(h * 16, 16)] * sp)
                        return 0
                    lax.fori_loop(0, SUB2, edge, 0)

                    pltpu.sync_copy(ev.at[pl.ds(0, SUB2)],
                                    expalpha.at[pl.ds(es, SUB2)])
                    pltpu.sync_copy(xwbuf, sh_out.at[dstv.at[j]], add=True)
                return 0
            lax.fori_loop(0, RQ, p2_sub, 0)
        return 0
    lax.fori_loop(0, IR2 // RB, p2_blk, 0)
    plsc.subcore_barrier()

    pltpu.sync_copy(sh_out.at[pl.ds(s * NZ, NZ)],
                    outacc.at[c].at[pl.ds(s * NZ, NZ)])

    @pl.when(s == 0)
    def _():
        pltpu.sync_copy(sh_out.at[pl.ds(NS * NZ, N - NS * NZ)],
                        outacc.at[c].at[pl.ds(NS * NZ, N - NS * NZ)])


_sc_edge = functools.partial(
    pl.kernel,
    out_type=(
        jax.ShapeDtypeStruct((EP, 16), jnp.float32),
        jax.ShapeDtypeStruct((NC, N, HC), jnp.float32),
    ),
    mesh=plsc.VectorSubcoreMesh(core_axis_name="c", subcore_axis_name="s"),
    compiler_params=pltpu.CompilerParams(use_tc_tiling_on_sc=False),
    scratch_types=[
        pltpu.VMEM_SHARED((N, HC), jnp.float32),
        pltpu.VMEM_SHARED((N, 16), jnp.float32),
        pltpu.VMEM_SHARED((N, 16), jnp.float32),
        pltpu.VMEM((RQ, 64), jnp.int32),
        pltpu.VMEM((RQ, 64), jnp.int32),
        pltpu.VMEM((SUB, 16), jnp.float32),
        pltpu.VMEM((SUB, 16), jnp.float32),
        pltpu.VMEM((SUB2, HC), jnp.float32),
        pltpu.VMEM((16,), jnp.float32),
    ],
)(_sc_body)


def kernel(x, edge_index, W, att_src, att_dst, bias):
    f32 = jnp.float32
    loops = jnp.arange(N, dtype=jnp.int32)
    pad = jnp.zeros((EP - E1,), dtype=jnp.int32)
    src = jnp.concatenate([edge_index[0], loops, pad]).reshape(EP // 64, 64)
    dst = jnp.concatenate([edge_index[1], loops, pad]).reshape(EP // 64, 64)

    eye = jnp.eye(H, dtype=f32)
    m_src = (att_src[:, :, None] * eye[:, None, :]).reshape(HC, H)
    m_dst = (att_dst[:, :, None] * eye[:, None, :]).reshape(HC, H)
    mcat = jnp.concatenate([m_src, m_dst], axis=1)
    p128 = jnp.tile(eye, (2, 16))

    xw, asd, srow = pl.pallas_call(
        _tc_prep,
        out_shape=(
            jax.ShapeDtypeStruct((N, HC), f32),
            jax.ShapeDtypeStruct((N, 16), f32),
            jax.ShapeDtypeStruct((1, 128), f32),
        ),
    )(x, W, mcat, p128)

    shift = srow.reshape(128)
    z128 = jnp.zeros((N, HC), f32)
    z16 = jnp.zeros((N, 16), f32)
    expalpha, outacc = _sc_edge(asd, xw, src, dst, shift, z128, z16)

    out = pl.pallas_call(
        _tc_final,
        out_shape=jax.ShapeDtypeStruct((N, HC), f32),
    )(outacc, x, bias.reshape(1, HC))

    return out, expalpha[:E1, :8]


# speedup vs baseline: 36.0692x; 1.2502x over previous
import functools

import jax
import jax.numpy as jnp
from jax import lax
from jax.experimental import pallas as pl
from jax.experimental.pallas import tpu as pltpu
from jax.experimental.pallas import tpu_sc as plsc

N = 10000
E = 320000
IN = 128
H = 8
C = 16
HC = H * C

E1 = E + N
NC, NS = 2, 16
NW = NC * NS

EP = 360448
T1 = EP // NS
T2 = EP // NW
IR1 = T1 // 128
IR2 = T2 // 128
RB = 8
RQ = 2 * RB
SUB = 128
SUB2 = 64
NZ = 624

_DNUMS = lax.GatherDimensionNumbers(
    offset_dims=(), collapsed_slice_dims=(0,), start_index_map=(0,))


def _dyngather(x, idx):
    return lax.gather(x, idx[:, None], _DNUMS, slice_sizes=(1,),
                      mode=lax.GatherScatterMode.PROMISE_IN_BOUNDS)


def _tc_prep(x_ref, w_ref, mcat_ref, p_ref, xw_ref, asd_ref, srow_ref):
    xw = jnp.dot(x_ref[...], w_ref[...], preferred_element_type=jnp.float32)
    xw_ref[...] = xw
    asd = jnp.dot(xw, mcat_ref[...], preferred_element_type=jnp.float32)
    asd_ref[...] = asd
    m = jnp.max(asd, axis=0, keepdims=True)
    s = jnp.dot(m, p_ref[...], preferred_element_type=jnp.float32)
    srow_ref[...] = jnp.maximum(s, 0.2 * s)


def _tc_final(acc_ref, x_ref, b_ref, out_ref):
    out_ref[...] = acc_ref[0] + acc_ref[1] + x_ref[...] + b_ref[...]


def _sc_body(asd, xw, srcq, dstq, srcp, dstp, shift, z128, z16,
             expalpha, outacc,
             sh_out, sh_den, sh_asd, srcv, dstv, srcw, dstw,
             svbuf, dvbuf, xwbuf, shiftv, sem1, sem2, sem3):
    c = lax.axis_index("c")
    s = lax.axis_index("s")
    w = s * NC + c

    lane = lax.iota(jnp.int32, 16)
    hi_idx = (lane & 7) + 8
    lo_maskf = jnp.clip(8.0 - lane.astype(jnp.float32), 0.0, 1.0)

    pltpu.sync_copy(z128.at[pl.ds(s * NZ, NZ)], sh_out.at[pl.ds(s * NZ, NZ)])
    pltpu.sync_copy(z16.at[pl.ds(s * NZ, NZ)], sh_den.at[pl.ds(s * NZ, NZ)])
    pltpu.sync_copy(asd.at[pl.ds(s * NZ, NZ)], sh_asd.at[pl.ds(s * NZ, NZ)])

    @pl.when(s == 0)
    def _():
        pltpu.sync_copy(z128.at[pl.ds(NS * NZ, N - NS * NZ)],
                        sh_out.at[pl.ds(NS * NZ, N - NS * NZ)])
        pltpu.sync_copy(z16.at[pl.ds(NS * NZ, N - NS * NZ)],
                        sh_den.at[pl.ds(NS * NZ, N - NS * NZ)])
        pltpu.sync_copy(asd.at[pl.ds(NS * NZ, N - NS * NZ)],
                        sh_asd.at[pl.ds(NS * NZ, N - NS * NZ)])

    pltpu.sync_copy(shift.at[pl.ds(0, 16)], shiftv)
    plsc.subcore_barrier()

    sv = shiftv[...]

    def p1_blk(ib, _):
        r0 = s * IR1 + ib * RB
        e0 = r0 * 128

        @pl.when(e0 < E1)
        def _():
            d1 = pltpu.async_copy(srcp.at[pl.ds(r0, RB)], srcv, sem1)
            d2 = pltpu.async_copy(dstp.at[pl.ds(r0, RB)], dstv, sem2)
            d3 = pltpu.async_copy(dstq.at[pl.ds(2 * r0, RQ)], dstw, sem3)
            d1.wait(); d2.wait(); d3.wait()

            def p1_sub(j, _):
                es = e0 + j * SUB

                @pl.when(es < E1)
                def _():
                    g1 = pltpu.async_copy(sh_asd.at[srcv.at[j]], svbuf, sem1)
                    g2 = pltpu.async_copy(sh_asd.at[dstv.at[j]], dvbuf, sem2)
                    g1.wait(); g2.wait()

                    def edge(e, _):
                        srow = svbuf[e, :]
                        drow = _dyngather(dvbuf[e, :], hi_idx)
                        t = srow + drow
                        t = jnp.maximum(t, 0.2 * t)
                        z = jnp.exp(t - sv)
                        padf = jnp.clip((E1 - (es + e)).astype(jnp.float32),
                                        0.0, 1.0)
                        svbuf[e, :] = z * (lo_maskf * padf)
                        return 0
                    lax.fori_loop(0, SUB, edge, 0)

                    w1 = pltpu.async_copy(svbuf, expalpha.at[pl.ds(es, SUB)],
                                          sem1)
                    w2 = pltpu.async_copy(svbuf.at[pl.ds(0, 64)],
                                          sh_den.at[dstw.at[2 * j]],
                                          sem2, add=True)
                    w3 = pltpu.async_copy(svbuf.at[pl.ds(64, 64)],
                                          sh_den.at[dstw.at[2 * j + 1]],
                                          sem3, add=True)
                    w1.wait(); w2.wait(); w3.wait()
                return 0
            lax.fori_loop(0, RB, p1_sub, 0)
        return 0
    lax.fori_loop(0, IR1 // RB, p1_blk, 0)
    plsc.subcore_barrier()

    ev, dbuf = svbuf, dvbuf

    def p2_blk(ib, _):
        r0 = w * IR2 + ib * RB
        e0 = r0 * 128

        @pl.when(e0 < E1)
        def _():
            d1 = pltpu.async_copy(srcp.at[pl.ds(r0, RB)], srcv, sem1)
            d2 = pltpu.async_copy(dstp.at[pl.ds(r0, RB)], dstv, sem2)
            d3 = pltpu.async_copy(dstq.at[pl.ds(2 * r0, RQ)], dstw, sem3)
            d1.wait(); d2.wait(); d3.wait()

            def p2_sub(j, _):
                es = e0 + j * SUB

                @pl.when(es < E1)
                def _():
                    g1 = pltpu.async_copy(expalpha.at[pl.ds(es, SUB)],
                                          ev, sem1)
                    g2 = pltpu.async_copy(xw.at[srcv.at[j]], xwbuf, sem2)
                    g3 = pltpu.async_copy(sh_den.at[dstv.at[j]], dbuf, sem3)
                    g1.wait(); g2.wait(); g3.wait()

                    def edge(e, _):
                        a = ev[e, :] / (dbuf[e, :] + 1e-16)
                        ev[e, :] = a
                        for h in range(H):
                            sp = _dyngather(a, jnp.full((16,), h, jnp.int32))
                            xwbuf[e, pl.ds(h * 16, 16)] = (
                                xwbuf[e, pl.ds(h * 16, 16)] * sp)
                        return 0
                    lax.fori_loop(0, SUB, edge, 0)

                    w1 = pltpu.async_copy(ev, expalpha.at[pl.ds(es, SUB)],
                                          sem1)
                    w2 = pltpu.async_copy(xwbuf.at[pl.ds(0, 64)],
                                          sh_out.at[dstw.at[2 * j]],
                                          sem2, add=True)
                    w3 = pltpu.async_copy(xwbuf.at[pl.ds(64, 64)],
                                          sh_out.at[dstw.at[2 * j + 1]],
                                          sem3, add=True)
                    w1.wait(); w2.wait(); w3.wait()
                return 0
            lax.fori_loop(0, RB, p2_sub, 0)
        return 0
    lax.fori_loop(0, IR2 // RB, p2_blk, 0)
    plsc.subcore_barrier()

    pltpu.sync_copy(sh_out.at[pl.ds(s * NZ, NZ)],
                    outacc.at[c].at[pl.ds(s * NZ, NZ)])

    @pl.when(s == 0)
    def _():
        pltpu.sync_copy(sh_out.at[pl.ds(NS * NZ, N - NS * NZ)],
                        outacc.at[c].at[pl.ds(NS * NZ, N - NS * NZ)])


_sc_edge = functools.partial(
    pl.kernel,
    out_type=(
        jax.ShapeDtypeStruct((EP, 16), jnp.float32),
        jax.ShapeDtypeStruct((NC, N, HC), jnp.float32),
    ),
    mesh=plsc.VectorSubcoreMesh(core_axis_name="c", subcore_axis_name="s"),
    compiler_params=pltpu.CompilerParams(use_tc_tiling_on_sc=False),
    scratch_types=[
        pltpu.VMEM_SHARED((N, HC), jnp.float32),
        pltpu.VMEM_SHARED((N, 16), jnp.float32),
        pltpu.VMEM_SHARED((N, 16), jnp.float32),
        pltpu.VMEM((RB, 128), jnp.int32),
        pltpu.VMEM((RB, 128), jnp.int32),
        pltpu.VMEM((RQ, 64), jnp.int32),
        pltpu.VMEM((RQ, 64), jnp.int32),
        pltpu.VMEM((SUB, 16), jnp.float32),
        pltpu.VMEM((SUB, 16), jnp.float32),
        pltpu.VMEM((SUB, HC), jnp.float32),
        pltpu.VMEM((16,), jnp.float32),
        pltpu.SemaphoreType.DMA,
        pltpu.SemaphoreType.DMA,
        pltpu.SemaphoreType.DMA,
    ],
)(_sc_body)


def kernel(x, edge_index, W, att_src, att_dst, bias):
    f32 = jnp.float32
    loops = jnp.arange(N, dtype=jnp.int32)
    pad = jnp.zeros((EP - E1,), dtype=jnp.int32)
    srcf = jnp.concatenate([edge_index[0], loops, pad])
    dstf = jnp.concatenate([edge_index[1], loops, pad])
    srcq, dstq = jax.lax.optimization_barrier(
        (srcf.reshape(EP // 64, 64), dstf.reshape(EP // 64, 64)))
    srcp = srcf.reshape(EP // 128, 128)
    dstp = dstf.reshape(EP // 128, 128)

    eye = jnp.eye(H, dtype=f32)
    m_src = (att_src[:, :, None] * eye[:, None, :]).reshape(HC, H)
    m_dst = (att_dst[:, :, None] * eye[:, None, :]).reshape(HC, H)
    mcat = jnp.concatenate([m_src, m_dst], axis=1)
    p128 = jnp.tile(eye, (2, 16))

    xw, asd, srow = pl.pallas_call(
        _tc_prep,
        out_shape=(
            jax.ShapeDtypeStruct((N, HC), f32),
            jax.ShapeDtypeStruct((N, 16), f32),
            jax.ShapeDtypeStruct((1, 128), f32),
        ),
    )(x, W, mcat, p128)

    shift = srow.reshape(128)
    z128 = jnp.zeros((N, HC), f32)
    z16 = jnp.zeros((N, 16), f32)
    expalpha, outacc = _sc_edge(asd, xw, srcq, dstq, srcp, dstp, shift,
                                z128, z16)

    out = pl.pallas_call(
        _tc_final,
        out_shape=jax.ShapeDtypeStruct((N, HC), f32),
    )(outacc, x, bias.reshape(1, HC))

    return out, expalpha[:E1, :8]


# speedup vs baseline: 57.6995x; 1.5997x over previous
import functools

import jax
import jax.numpy as jnp
from jax import lax
from jax.experimental import pallas as pl
from jax.experimental.pallas import tpu as pltpu
from jax.experimental.pallas import tpu_sc as plsc

N = 10000
E = 320000
IN = 128
H = 8
C = 16
HC = H * C

E1 = E + N
NC, NS = 2, 16
NW = NC * NS

EP = 360448
T1 = EP // NS
T2 = EP // NW
IR1 = T1 // 128
IR2 = T2 // 128
RB = 8
RQ = 2 * RB
SUB = 128
SUB2 = 64
NZ = 624

_DNUMS = lax.GatherDimensionNumbers(
    offset_dims=(), collapsed_slice_dims=(0,), start_index_map=(0,))


def _dyngather(x, idx):
    return lax.gather(x, idx[:, None], _DNUMS, slice_sizes=(1,),
                      mode=lax.GatherScatterMode.PROMISE_IN_BOUNDS)


def _tc_prep(x_ref, w_ref, mcat_ref, p_ref, xw_ref, asd_ref, srow_ref):
    xw = jnp.dot(x_ref[...], w_ref[...], preferred_element_type=jnp.float32)
    xw_ref[...] = xw
    asd = jnp.dot(xw, mcat_ref[...], preferred_element_type=jnp.float32)
    asd_ref[...] = asd
    m = jnp.max(asd, axis=0, keepdims=True)
    s = jnp.dot(m, p_ref[...], preferred_element_type=jnp.float32)
    srow_ref[...] = jnp.maximum(s, 0.2 * s)


def _tc_final(acc_ref, x_ref, b_ref, out_ref):
    out_ref[...] = acc_ref[0] + acc_ref[1] + x_ref[...] + b_ref[...]


def _sc_body(asd, xw, srcq, dstq, srcp, dstp, shift, z128, z16,
             expalpha, outacc,
             sh_out, sh_den, sh_asd, srcv, dstv, srcw, dstw,
             svbuf, dvbuf, xwbuf, shiftv, sem1, sem2, sem3):
    c = lax.axis_index("c")
    s = lax.axis_index("s")
    w = s * NC + c

    lane = lax.iota(jnp.int32, 16)
    hi_idx = (lane & 7) + 8
    lo_maskf = jnp.clip(8.0 - lane.astype(jnp.float32), 0.0, 1.0)

    pltpu.sync_copy(z128.at[pl.ds(s * NZ, NZ)], sh_out.at[pl.ds(s * NZ, NZ)])
    pltpu.sync_copy(z16.at[pl.ds(s * NZ, NZ)], sh_den.at[pl.ds(s * NZ, NZ)])
    pltpu.sync_copy(asd.at[pl.ds(s * NZ, NZ)], sh_asd.at[pl.ds(s * NZ, NZ)])

    @pl.when(s == 0)
    def _():
        pltpu.sync_copy(z128.at[pl.ds(NS * NZ, N - NS * NZ)],
                        sh_out.at[pl.ds(NS * NZ, N - NS * NZ)])
        pltpu.sync_copy(z16.at[pl.ds(NS * NZ, N - NS * NZ)],
                        sh_den.at[pl.ds(NS * NZ, N - NS * NZ)])
        pltpu.sync_copy(asd.at[pl.ds(NS * NZ, N - NS * NZ)],
                        sh_asd.at[pl.ds(NS * NZ, N - NS * NZ)])

    pltpu.sync_copy(shift.at[pl.ds(0, 16)], shiftv)
    plsc.subcore_barrier()

    sv = shiftv[...]

    def p1_blk(ib, _):
        r0 = s * IR1 + ib * RB
        e0 = r0 * 128

        @pl.when(e0 < E1)
        def _():
            d1 = pltpu.async_copy(srcp.at[pl.ds(r0, RB)], srcv, sem1)
            d2 = pltpu.async_copy(dstp.at[pl.ds(r0, RB)], dstv, sem2)
            d3 = pltpu.async_copy(dstq.at[pl.ds(2 * r0, RQ)], dstw, sem3)
            d1.wait(); d2.wait(); d3.wait()

            def p1_sub(j, _):
                es = e0 + j * SUB

                @pl.when(es < E1)
                def _():
                    g1 = pltpu.async_copy(sh_asd.at[srcv.at[j]], svbuf, sem1)
                    g2 = pltpu.async_copy(sh_asd.at[dstv.at[j]], dvbuf, sem2)
                    g1.wait(); g2.wait()

                    @plsc.parallel_loop(0, SUB, unroll=4)
                    def _(e):
                        srow = svbuf[e, :]
                        drow = _dyngather(dvbuf[e, :], hi_idx)
                        t = srow + drow
                        t = jnp.maximum(t, 0.2 * t)
                        z = jnp.exp(t - sv)
                        padf = jnp.clip((E1 - (es + e)).astype(jnp.float32),
                                        0.0, 1.0)
                        svbuf[e, :] = z * (lo_maskf * padf)

                    w1 = pltpu.async_copy(svbuf, expalpha.at[pl.ds(es, SUB)],
                                          sem1)
                    w2 = pltpu.async_copy(svbuf.at[pl.ds(0, 64)],
                                          sh_den.at[dstw.at[2 * j]],
                                          sem2, add=True)
                    w3 = pltpu.async_copy(svbuf.at[pl.ds(64, 64)],
                                          sh_den.at[dstw.at[2 * j + 1]],
                                          sem3, add=True)
                    w1.wait(); w2.wait(); w3.wait()
                return 0
            lax.fori_loop(0, RB, p1_sub, 0)
        return 0
    lax.fori_loop(0, IR1 // RB, p1_blk, 0)
    plsc.subcore_barrier()

    ev, dbuf = svbuf, dvbuf

    def p2_blk(ib, _):
        r0 = w * IR2 + ib * RB
        e0 = r0 * 128

        @pl.when(e0 < E1)
        def _():
            d1 = pltpu.async_copy(srcp.at[pl.ds(r0, RB)], srcv, sem1)
            d2 = pltpu.async_copy(dstp.at[pl.ds(r0, RB)], dstv, sem2)
            d3 = pltpu.async_copy(dstq.at[pl.ds(2 * r0, RQ)], dstw, sem3)
            d1.wait(); d2.wait(); d3.wait()

            def p2_sub(j, _):
                es = e0 + j * SUB

                @pl.when(es < E1)
                def _():
                    g1 = pltpu.async_copy(expalpha.at[pl.ds(es, SUB)],
                                          ev, sem1)
                    g2 = pltpu.async_copy(xw.at[srcv.at[j]], xwbuf, sem2)
                    g3 = pltpu.async_copy(sh_den.at[dstv.at[j]], dbuf, sem3)
                    g1.wait(); g2.wait(); g3.wait()

                    @plsc.parallel_loop(0, SUB, unroll=2)
                    def _(e):
                        a = ev[e, :] / (dbuf[e, :] + 1e-16)
                        ev[e, :] = a
                        for h in range(H):
                            sp = jnp.full((16,), a[h], jnp.float32)
                            xwbuf[e, pl.ds(h * 16, 16)] = (
                                xwbuf[e, pl.ds(h * 16, 16)] * sp)

                    w1 = pltpu.async_copy(ev, expalpha.at[pl.ds(es, SUB)],
                                          sem1)
                    w2 = pltpu.async_copy(xwbuf.at[pl.ds(0, 64)],
                                          sh_out.at[dstw.at[2 * j]],
                                          sem2, add=True)
                    w3 = pltpu.async_copy(xwbuf.at[pl.ds(64, 64)],
                                          sh_out.at[dstw.at[2 * j + 1]],
                                          sem3, add=True)
                    w1.wait(); w2.wait(); w3.wait()
                return 0
            lax.fori_loop(0, RB, p2_sub, 0)
        return 0
    lax.fori_loop(0, IR2 // RB, p2_blk, 0)
    plsc.subcore_barrier()

    pltpu.sync_copy(sh_out.at[pl.ds(s * NZ, NZ)],
                    outacc.at[c].at[pl.ds(s * NZ, NZ)])

    @pl.when(s == 0)
    def _():
        pltpu.sync_copy(sh_out.at[pl.ds(NS * NZ, N - NS * NZ)],
                        outacc.at[c].at[pl.ds(NS * NZ, N - NS * NZ)])


_sc_edge = functools.partial(
    pl.kernel,
    out_type=(
        jax.ShapeDtypeStruct((EP, 16), jnp.float32),
        jax.ShapeDtypeStruct((NC, N, HC), jnp.float32),
    ),
    mesh=plsc.VectorSubcoreMesh(core_axis_name="c", subcore_axis_name="s"),
    compiler_params=pltpu.CompilerParams(use_tc_tiling_on_sc=False),
    scratch_types=[
        pltpu.VMEM_SHARED((N, HC), jnp.float32),
        pltpu.VMEM_SHARED((N, 16), jnp.float32),
        pltpu.VMEM_SHARED((N, 16), jnp.float32),
        pltpu.VMEM((RB, 128), jnp.int32),
        pltpu.VMEM((RB, 128), jnp.int32),
        pltpu.VMEM((RQ, 64), jnp.int32),
        pltpu.VMEM((RQ, 64), jnp.int32),
        pltpu.VMEM((SUB, 16), jnp.float32),
        pltpu.VMEM((SUB, 16), jnp.float32),
        pltpu.VMEM((SUB, HC), jnp.float32),
        pltpu.VMEM((16,), jnp.float32),
        pltpu.SemaphoreType.DMA,
        pltpu.SemaphoreType.DMA,
        pltpu.SemaphoreType.DMA,
    ],
)(_sc_body)


def kernel(x, edge_index, W, att_src, att_dst, bias):
    f32 = jnp.float32
    loops = jnp.arange(N, dtype=jnp.int32)
    pad = jnp.zeros((EP - E1,), dtype=jnp.int32)
    srcf = jnp.concatenate([edge_index[0], loops, pad])
    dstf = jnp.concatenate([edge_index[1], loops, pad])
    srcq, dstq = jax.lax.optimization_barrier(
        (srcf.reshape(EP // 64, 64), dstf.reshape(EP // 64, 64)))
    srcp = srcf.reshape(EP // 128, 128)
    dstp = dstf.reshape(EP // 128, 128)

    eye = jnp.eye(H, dtype=f32)
    m_src = (att_src[:, :, None] * eye[:, None, :]).reshape(HC, H)
    m_dst = (att_dst[:, :, None] * eye[:, None, :]).reshape(HC, H)
    mcat = jnp.concatenate([m_src, m_dst], axis=1)
    p128 = jnp.tile(eye, (2, 16))

    xw, asd, srow = pl.pallas_call(
        _tc_prep,
        out_shape=(
            jax.ShapeDtypeStruct((N, HC), f32),
            jax.ShapeDtypeStruct((N, 16), f32),
            jax.ShapeDtypeStruct((1, 128), f32),
        ),
    )(x, W, mcat, p128)

    shift = srow.reshape(128)
    z128 = jnp.zeros((N, HC), f32)
    z16 = jnp.zeros((N, 16), f32)
    expalpha, outacc = _sc_edge(asd, xw, srcq, dstq, srcp, dstp, shift,
                                z128, z16)

    out = pl.pallas_call(
        _tc_final,
        out_shape=jax.ShapeDtypeStruct((N, HC), f32),
    )(outacc, x, bias.reshape(1, HC))

    return out, expalpha[:E1, :8]


# speedup vs baseline: 57.9498x; 1.0043x over previous
import functools

import jax
import jax.numpy as jnp
from jax import lax
from jax.experimental import pallas as pl
from jax.experimental.pallas import tpu as pltpu
from jax.experimental.pallas import tpu_sc as plsc

N = 10000
E = 320000
IN = 128
H = 8
C = 16
HC = H * C

E1 = E + N
NC, NS = 2, 16
NW = NC * NS

EP = 360448
T1 = EP // NS
T2 = EP // NW
IR1 = T1 // 128
IR2 = T2 // 128
RB = 8
RQ = 2 * RB
SUB = 128
SUB2 = 64
NZ = 624

_DNUMS = lax.GatherDimensionNumbers(
    offset_dims=(), collapsed_slice_dims=(0,), start_index_map=(0,))


def _dyngather(x, idx):
    return lax.gather(x, idx[:, None], _DNUMS, slice_sizes=(1,),
                      mode=lax.GatherScatterMode.PROMISE_IN_BOUNDS)


def _tc_prep(x_ref, w_ref, mcat_ref, p_ref, xw_ref, asd_ref, srow_ref):
    xw = jnp.dot(x_ref[...], w_ref[...], preferred_element_type=jnp.float32)
    xw_ref[...] = xw
    asd = jnp.dot(xw, mcat_ref[...], preferred_element_type=jnp.float32)
    asd_ref[...] = asd
    m = jnp.max(asd, axis=0, keepdims=True)
    s = jnp.dot(m, p_ref[...], preferred_element_type=jnp.float32)
    srow_ref[...] = jnp.maximum(s, 0.2 * s)


def _tc_final(acc_ref, x_ref, b_ref, out_ref):
    out_ref[...] = acc_ref[0] + acc_ref[1] + x_ref[...] + b_ref[...]


def _sc_body(asd, xw, srcq, dstq, srcp, dstp, shift, z128, z16,
             expalpha, outacc,
             sh_out, sh_den, sh_asd, srcv, dstv, srcw, dstw,
             svbuf, dvbuf, xwbuf, shiftv, sem1, sem2, sem3):
    c = lax.axis_index("c")
    s = lax.axis_index("s")
    w = s * NC + c

    lane = lax.iota(jnp.int32, 16)
    hi_idx = (lane & 7) + 8
    lo_maskf = jnp.clip(8.0 - lane.astype(jnp.float32), 0.0, 1.0)

    pltpu.sync_copy(z128.at[pl.ds(s * NZ, NZ)], sh_out.at[pl.ds(s * NZ, NZ)])
    pltpu.sync_copy(z16.at[pl.ds(s * NZ, NZ)], sh_den.at[pl.ds(s * NZ, NZ)])
    pltpu.sync_copy(asd.at[pl.ds(s * NZ, NZ)], sh_asd.at[pl.ds(s * NZ, NZ)])

    @pl.when(s == 0)
    def _():
        pltpu.sync_copy(z128.at[pl.ds(NS * NZ, N - NS * NZ)],
                        sh_out.at[pl.ds(NS * NZ, N - NS * NZ)])
        pltpu.sync_copy(z16.at[pl.ds(NS * NZ, N - NS * NZ)],
                        sh_den.at[pl.ds(NS * NZ, N - NS * NZ)])
        pltpu.sync_copy(asd.at[pl.ds(NS * NZ, N - NS * NZ)],
                        sh_asd.at[pl.ds(NS * NZ, N - NS * NZ)])

    pltpu.sync_copy(shift.at[pl.ds(0, 16)], shiftv)
    plsc.subcore_barrier()

    sv = shiftv[...]

    def p1_blk(ib, _):
        r0 = s * IR1 + ib * RB
        e0 = r0 * 128

        @pl.when(e0 < E1)
        def _():
            d1 = pltpu.async_copy(srcp.at[pl.ds(r0, RB)], srcv, sem1)
            d2 = pltpu.async_copy(dstp.at[pl.ds(r0, RB)], dstv, sem2)
            d3 = pltpu.async_copy(dstq.at[pl.ds(2 * r0, RQ)], dstw, sem3)
            d1.wait(); d2.wait(); d3.wait()

            def p1_sub(j, _):
                es = e0 + j * SUB

                @pl.when(es < E1)
                def _():
                    g1 = pltpu.async_copy(sh_asd.at[srcv.at[j]], svbuf, sem1)
                    g2 = pltpu.async_copy(sh_asd.at[dstv.at[j]], dvbuf, sem2)
                    g1.wait(); g2.wait()

                    @plsc.parallel_loop(0, SUB, unroll=8)
                    def _(e):
                        srow = svbuf[e, :]
                        drow = _dyngather(dvbuf[e, :], hi_idx)
                        t = srow + drow
                        t = jnp.maximum(t, 0.2 * t)
                        z = jnp.exp(t - sv)
                        padf = jnp.clip((E1 - (es + e)).astype(jnp.float32),
                                        0.0, 1.0)
                        svbuf[e, :] = z * (lo_maskf * padf)

                    w1 = pltpu.async_copy(svbuf, expalpha.at[pl.ds(es, SUB)],
                                          sem1)
                    w2 = pltpu.async_copy(svbuf.at[pl.ds(0, 64)],
                                          sh_den.at[dstw.at[2 * j]],
                                          sem2, add=True)
                    w3 = pltpu.async_copy(svbuf.at[pl.ds(64, 64)],
                                          sh_den.at[dstw.at[2 * j + 1]],
                                          sem3, add=True)
                    w1.wait(); w2.wait(); w3.wait()
                return 0
            lax.fori_loop(0, RB, p1_sub, 0)
        return 0
    lax.fori_loop(0, IR1 // RB, p1_blk, 0)
    plsc.subcore_barrier()

    ev, dbuf = svbuf, dvbuf

    def p2_blk(ib, _):
        r0 = w * IR2 + ib * RB
        e0 = r0 * 128

        @pl.when(e0 < E1)
        def _():
            d1 = pltpu.async_copy(srcp.at[pl.ds(r0, RB)], srcv, sem1)
            d2 = pltpu.async_copy(dstp.at[pl.ds(r0, RB)], dstv, sem2)
            d3 = pltpu.async_copy(dstq.at[pl.ds(2 * r0, RQ)], dstw, sem3)
            d1.wait(); d2.wait(); d3.wait()

            def p2_sub(j, _):
                es = e0 + j * SUB

                @pl.when(es < E1)
                def _():
                    g1 = pltpu.async_copy(expalpha.at[pl.ds(es, SUB)],
                                          ev, sem1)
                    g2 = pltpu.async_copy(xw.at[srcv.at[j]], xwbuf, sem2)
                    g3 = pltpu.async_copy(sh_den.at[dstv.at[j]], dbuf, sem3)
                    g1.wait(); g2.wait(); g3.wait()

                    @plsc.parallel_loop(0, SUB, unroll=4)
                    def _(e):
                        a = ev[e, :] / (dbuf[e, :] + 1e-16)
                        ev[e, :] = a
                        for h in range(H):
                            sp = jnp.full((16,), a[h], jnp.float32)
                            xwbuf[e, pl.ds(h * 16, 16)] = (
                                xwbuf[e, pl.ds(h * 16, 16)] * sp)

                    w1 = pltpu.async_copy(ev, expalpha.at[pl.ds(es, SUB)],
                                          sem1)
                    w2 = pltpu.async_copy(xwbuf.at[pl.ds(0, 64)],
                                          sh_out.at[dstw.at[2 * j]],
                                          sem2, add=True)
                    w3 = pltpu.async_copy(xwbuf.at[pl.ds(64, 64)],
                                          sh_out.at[dstw.at[2 * j + 1]],
                                          sem3, add=True)
                    w1.wait(); w2.wait(); w3.wait()
                return 0
            lax.fori_loop(0, RB, p2_sub, 0)
        return 0
    lax.fori_loop(0, IR2 // RB, p2_blk, 0)
    plsc.subcore_barrier()

    pltpu.sync_copy(sh_out.at[pl.ds(s * NZ, NZ)],
                    outacc.at[c].at[pl.ds(s * NZ, NZ)])

    @pl.when(s == 0)
    def _():
        pltpu.sync_copy(sh_out.at[pl.ds(NS * NZ, N - NS * NZ)],
                        outacc.at[c].at[pl.ds(NS * NZ, N - NS * NZ)])


_sc_edge = functools.partial(
    pl.kernel,
    out_type=(
        jax.ShapeDtypeStruct((EP, 16), jnp.float32),
        jax.ShapeDtypeStruct((NC, N, HC), jnp.float32),
    ),
    mesh=plsc.VectorSubcoreMesh(core_axis_name="c", subcore_axis_name="s"),
    compiler_params=pltpu.CompilerParams(use_tc_tiling_on_sc=False),
    scratch_types=[
        pltpu.VMEM_SHARED((N, HC), jnp.float32),
        pltpu.VMEM_SHARED((N, 16), jnp.float32),
        pltpu.VMEM_SHARED((N, 16), jnp.float32),
        pltpu.VMEM((RB, 128), jnp.int32),
        pltpu.VMEM((RB, 128), jnp.int32),
        pltpu.VMEM((RQ, 64), jnp.int32),
        pltpu.VMEM((RQ, 64), jnp.int32),
        pltpu.VMEM((SUB, 16), jnp.float32),
        pltpu.VMEM((SUB, 16), jnp.float32),
        pltpu.VMEM((SUB, HC), jnp.float32),
        pltpu.VMEM((16,), jnp.float32),
        pltpu.SemaphoreType.DMA,
        pltpu.SemaphoreType.DMA,
        pltpu.SemaphoreType.DMA,
    ],
)(_sc_body)


def kernel(x, edge_index, W, att_src, att_dst, bias):
    f32 = jnp.float32
    loops = jnp.arange(N, dtype=jnp.int32)
    pad = jnp.zeros((EP - E1,), dtype=jnp.int32)
    srcf = jnp.concatenate([edge_index[0], loops, pad])
    dstf = jnp.concatenate([edge_index[1], loops, pad])
    srcq, dstq = jax.lax.optimization_barrier(
        (srcf.reshape(EP // 64, 64), dstf.reshape(EP // 64, 64)))
    srcp = srcf.reshape(EP // 128, 128)
    dstp = dstf.reshape(EP // 128, 128)

    eye = jnp.eye(H, dtype=f32)
    m_src = (att_src[:, :, None] * eye[:, None, :]).reshape(HC, H)
    m_dst = (att_dst[:, :, None] * eye[:, None, :]).reshape(HC, H)
    mcat = jnp.concatenate([m_src, m_dst], axis=1)
    p128 = jnp.tile(eye, (2, 16))

    xw, asd, srow = pl.pallas_call(
        _tc_prep,
        out_shape=(
            jax.ShapeDtypeStruct((N, HC), f32),
            jax.ShapeDtypeStruct((N, 16), f32),
            jax.ShapeDtypeStruct((1, 128), f32),
        ),
    )(x, W, mcat, p128)

    shift = srow.reshape(128)
    z128 = jnp.zeros((N, HC), f32)
    z16 = jnp.zeros((N, 16), f32)
    expalpha, outacc = _sc_edge(asd, xw, srcq, dstq, srcp, dstp, shift,
                                z128, z16)

    out = pl.pallas_call(
        _tc_final,
        out_shape=jax.ShapeDtypeStruct((N, HC), f32),
    )(outacc, x, bias.reshape(1, HC))

    return out, expalpha[:E1, :8]


# speedup vs baseline: 58.2476x; 1.0051x over previous
import functools

import jax
import jax.numpy as jnp
from jax import lax
from jax.experimental import pallas as pl
from jax.experimental.pallas import tpu as pltpu
from jax.experimental.pallas import tpu_sc as plsc

N = 10000
E = 320000
IN = 128
H = 8
C = 16
HC = H * C

E1 = E + N
NC, NS = 2, 16
NW = NC * NS

EP = 360448
T1 = EP // NS
T2 = EP // NW
IR1 = T1 // 128
IR2 = T2 // 128
RB = 8
RQ = 2 * RB
SUB = 128
SUB2 = 64
NZ = 624

_DNUMS = lax.GatherDimensionNumbers(
    offset_dims=(), collapsed_slice_dims=(0,), start_index_map=(0,))


def _dyngather(x, idx):
    return lax.gather(x, idx[:, None], _DNUMS, slice_sizes=(1,),
                      mode=lax.GatherScatterMode.PROMISE_IN_BOUNDS)


def _tc_prep(x_ref, w_ref, mcat_ref, p_ref, xw_ref, asd_ref, srow_ref):
    xw = jnp.dot(x_ref[...], w_ref[...], preferred_element_type=jnp.float32)
    xw_ref[...] = xw
    asd = jnp.dot(xw, mcat_ref[...], preferred_element_type=jnp.float32)
    asd_ref[...] = asd
    m = jnp.max(asd, axis=0, keepdims=True)
    s = jnp.dot(m, p_ref[...], preferred_element_type=jnp.float32)
    srow_ref[...] = jnp.maximum(s, 0.2 * s)


def _tc_final(acc_ref, x_ref, b_ref, out_ref):
    out_ref[...] = acc_ref[0] + acc_ref[1] + x_ref[...] + b_ref[...]


def _sc_body(asd, xw, srcp, dstp, shift, z128, z16,
             expalpha, outacc,
             sh_out, sh_den, sh_asd, srcv, dstv,
             svbuf, dvbuf, xwbuf, shiftv, sem1, sem2, sem3):
    c = lax.axis_index("c")
    s = lax.axis_index("s")
    w = s * NC + c

    lane = lax.iota(jnp.int32, 16)
    hi_idx = (lane & 7) + 8
    lo_maskf = jnp.clip(8.0 - lane.astype(jnp.float32), 0.0, 1.0)

    pltpu.sync_copy(z128.at[pl.ds(s * NZ, NZ)], sh_out.at[pl.ds(s * NZ, NZ)])
    pltpu.sync_copy(z16.at[pl.ds(s * NZ, NZ)], sh_den.at[pl.ds(s * NZ, NZ)])
    pltpu.sync_copy(asd.at[pl.ds(s * NZ, NZ)], sh_asd.at[pl.ds(s * NZ, NZ)])

    @pl.when(s == 0)
    def _():
        pltpu.sync_copy(z128.at[pl.ds(NS * NZ, N - NS * NZ)],
                        sh_out.at[pl.ds(NS * NZ, N - NS * NZ)])
        pltpu.sync_copy(z16.at[pl.ds(NS * NZ, N - NS * NZ)],
                        sh_den.at[pl.ds(NS * NZ, N - NS * NZ)])
        pltpu.sync_copy(asd.at[pl.ds(NS * NZ, N - NS * NZ)],
                        sh_asd.at[pl.ds(NS * NZ, N - NS * NZ)])

    pltpu.sync_copy(shift.at[pl.ds(0, 16)], shiftv)
    plsc.subcore_barrier()

    sv = shiftv[...]

    def p1_blk(ib, _):
        r0 = s * IR1 + ib * RB
        e0 = r0 * 128

        @pl.when(e0 < E1)
        def _():
            d1 = pltpu.async_copy(srcp.at[pl.ds(r0, RB)], srcv, sem1)
            d2 = pltpu.async_copy(dstp.at[pl.ds(r0, RB)], dstv, sem2)
            d1.wait(); d2.wait()

            def p1_sub(j, _):
                es = e0 + j * SUB

                @pl.when(es < E1)
                def _():
                    g1 = pltpu.async_copy(sh_asd.at[srcv.at[j]], svbuf, sem1)
                    g2 = pltpu.async_copy(sh_asd.at[dstv.at[j]], dvbuf, sem2)
                    g1.wait(); g2.wait()

                    @plsc.parallel_loop(0, SUB, unroll=8)
                    def _(e):
                        srow = svbuf[e, :]
                        drow = _dyngather(dvbuf[e, :], hi_idx)
                        t = srow + drow
                        t = jnp.maximum(t, 0.2 * t)
                        z = jnp.exp(t - sv)
                        padf = jnp.clip((E1 - (es + e)).astype(jnp.float32),
                                        0.0, 1.0)
                        svbuf[e, :] = z * (lo_maskf * padf)

                    w1 = pltpu.async_copy(svbuf, expalpha.at[pl.ds(es, SUB)],
                                          sem1)
                    w2 = pltpu.async_copy(svbuf, sh_den.at[dstv.at[j]],
                                          sem2, add=True)
                    w1.wait(); w2.wait()
                return 0
            lax.fori_loop(0, RB, p1_sub, 0)
        return 0
    lax.fori_loop(0, IR1 // RB, p1_blk, 0)
    plsc.subcore_barrier()

    ev, dbuf = svbuf, dvbuf

    def p2_blk(ib, _):
        r0 = w * IR2 + ib * RB
        e0 = r0 * 128

        @pl.when(e0 < E1)
        def _():
            d1 = pltpu.async_copy(srcp.at[pl.ds(r0, RB)], srcv, sem1)
            d2 = pltpu.async_copy(dstp.at[pl.ds(r0, RB)], dstv, sem2)
            d1.wait(); d2.wait()

            def p2_sub(j, _):
                es = e0 + j * SUB

                @pl.when(es < E1)
                def _():
                    g1 = pltpu.async_copy(expalpha.at[pl.ds(es, SUB)],
                                          ev, sem1)
                    g2 = pltpu.async_copy(xw.at[srcv.at[j]], xwbuf, sem2)
                    g3 = pltpu.async_copy(sh_den.at[dstv.at[j]], dbuf, sem3)
                    g1.wait(); g2.wait(); g3.wait()

                    @plsc.parallel_loop(0, SUB, unroll=4)
                    def _(e):
                        a = ev[e, :] / (dbuf[e, :] + 1e-16)
                        ev[e, :] = a
                        for h in range(H):
                            sp = jnp.full((16,), a[h], jnp.float32)
                            xwbuf[e, pl.ds(h * 16, 16)] = (
                                xwbuf[e, pl.ds(h * 16, 16)] * sp)

                    w1 = pltpu.async_copy(ev, expalpha.at[pl.ds(es, SUB)],
                                          sem1)
                    w2 = pltpu.async_copy(xwbuf, sh_out.at[dstv.at[j]],
                                          sem2, add=True)
                    w1.wait(); w2.wait()
                return 0
            lax.fori_loop(0, RB, p2_sub, 0)
        return 0
    lax.fori_loop(0, IR2 // RB, p2_blk, 0)
    plsc.subcore_barrier()

    pltpu.sync_copy(sh_out.at[pl.ds(s * NZ, NZ)],
                    outacc.at[c].at[pl.ds(s * NZ, NZ)])

    @pl.when(s == 0)
    def _():
        pltpu.sync_copy(sh_out.at[pl.ds(NS * NZ, N - NS * NZ)],
                        outacc.at[c].at[pl.ds(NS * NZ, N - NS * NZ)])


_sc_edge = functools.partial(
    pl.kernel,
    out_type=(
        jax.ShapeDtypeStruct((EP, 16), jnp.float32),
        jax.ShapeDtypeStruct((NC, N, HC), jnp.float32),
    ),
    mesh=plsc.VectorSubcoreMesh(core_axis_name="c", subcore_axis_name="s"),
    compiler_params=pltpu.CompilerParams(use_tc_tiling_on_sc=False),
    scratch_types=[
        pltpu.VMEM_SHARED((N, HC), jnp.float32),
        pltpu.VMEM_SHARED((N, 16), jnp.float32),
        pltpu.VMEM_SHARED((N, 16), jnp.float32),
        pltpu.VMEM((RB, 128), jnp.int32),
        pltpu.VMEM((RB, 128), jnp.int32),
        pltpu.VMEM((SUB, 16), jnp.float32),
        pltpu.VMEM((SUB, 16), jnp.float32),
        pltpu.VMEM((SUB, HC), jnp.float32),
        pltpu.VMEM((16,), jnp.float32),
        pltpu.SemaphoreType.DMA,
        pltpu.SemaphoreType.DMA,
        pltpu.SemaphoreType.DMA,
    ],
)(_sc_body)


def kernel(x, edge_index, W, att_src, att_dst, bias):
    f32 = jnp.float32
    loops = jnp.arange(N, dtype=jnp.int32)
    pad = jnp.zeros((EP - E1,), dtype=jnp.int32)
    srcf = jnp.concatenate([edge_index[0], loops, pad])
    dstf = jnp.concatenate([edge_index[1], loops, pad])
    srcp = srcf.reshape(EP // 128, 128)
    dstp = dstf.reshape(EP // 128, 128)

    eye = jnp.eye(H, dtype=f32)
    m_src = (att_src[:, :, None] * eye[:, None, :]).reshape(HC, H)
    m_dst = (att_dst[:, :, None] * eye[:, None, :]).reshape(HC, H)
    mcat = jnp.concatenate([m_src, m_dst], axis=1)
    p128 = jnp.tile(eye, (2, 16))

    xw, asd, srow = pl.pallas_call(
        _tc_prep,
        out_shape=(
            jax.ShapeDtypeStruct((N, HC), f32),
            jax.ShapeDtypeStruct((N, 16), f32),
            jax.ShapeDtypeStruct((1, 128), f32),
        ),
    )(x, W, mcat, p128)

    shift = srow.reshape(128)
    z128 = jnp.zeros((N, HC), f32)
    z16 = jnp.zeros((N, 16), f32)
    expalpha, outacc = _sc_edge(asd, xw, srcp, dstp, shift, z128, z16)

    out = pl.pallas_call(
        _tc_final,
        out_shape=jax.ShapeDtypeStruct((N, HC), f32),
    )(outacc, x, bias.reshape(1, HC))

    return out, expalpha[:E1, :8]


# speedup vs baseline: 58.8819x; 1.0109x over previous
import functools

import jax
import jax.numpy as jnp
from jax import lax
from jax.experimental import pallas as pl
from jax.experimental.pallas import tpu as pltpu
from jax.experimental.pallas import tpu_sc as plsc

N = 10000
E = 320000
IN = 128
H = 8
C = 16
HC = H * C

E1 = E + N
NC, NS = 2, 16
NW = NC * NS

EP = 360448
T1 = EP // NS
T2 = EP // NW
IR1 = T1 // 128
IR2 = T2 // 128
RB = 8
RQ = 2 * RB
SUB = 128
SUB2 = 64
NZ = 624

_DNUMS = lax.GatherDimensionNumbers(
    offset_dims=(), collapsed_slice_dims=(0,), start_index_map=(0,))


def _dyngather(x, idx):
    return lax.gather(x, idx[:, None], _DNUMS, slice_sizes=(1,),
                      mode=lax.GatherScatterMode.PROMISE_IN_BOUNDS)


def _tc_prep(x_ref, w_ref, mcat_ref, p_ref, xw_ref, asd_ref, srow_ref):
    xw = jnp.dot(x_ref[...], w_ref[...], preferred_element_type=jnp.float32)
    xw_ref[...] = xw
    asd = jnp.dot(xw, mcat_ref[...], preferred_element_type=jnp.float32)
    asd_ref[...] = asd
    m = jnp.max(asd, axis=0, keepdims=True)
    s = jnp.dot(m, p_ref[...], preferred_element_type=jnp.float32)
    srow_ref[...] = jnp.maximum(s, 0.2 * s)


def _tc_final(acc_ref, x_ref, b_ref, out_ref):
    out_ref[...] = acc_ref[0] + acc_ref[1] + x_ref[...] + b_ref[...]


def _sc_body(asd, xw, srcp, dstp, shift,
             expalpha, outacc,
             sh_out, sh_den, sh_asd, srcv, dstv,
             svbuf, dvbuf, xwbuf, shiftv, sem1, sem2, sem3):
    c = lax.axis_index("c")
    s = lax.axis_index("s")
    w = s * NC + c

    lane = lax.iota(jnp.int32, 16)
    hi_idx = (lane & 7) + 8
    lo_maskf = jnp.clip(8.0 - lane.astype(jnp.float32), 0.0, 1.0)

    @plsc.parallel_loop(0, SUB)
    def _(r):
        zrow = jnp.zeros((16,), jnp.float32)
        for q in range(HC // 16):
            xwbuf[r, pl.ds(q * 16, 16)] = zrow
        svbuf[r, :] = zrow

    for q in range(5):
        pltpu.sync_copy(xwbuf.at[pl.ds(0, 125)],
                        sh_out.at[pl.ds(s * 625 + q * 125, 125)])
        pltpu.sync_copy(svbuf.at[pl.ds(0, 125)],
                        sh_den.at[pl.ds(s * 625 + q * 125, 125)])
    pltpu.sync_copy(asd.at[pl.ds(s * NZ, NZ)], sh_asd.at[pl.ds(s * NZ, NZ)])

    @pl.when(s == 0)
    def _():
        pltpu.sync_copy(asd.at[pl.ds(NS * NZ, N - NS * NZ)],
                        sh_asd.at[pl.ds(NS * NZ, N - NS * NZ)])

    pltpu.sync_copy(shift.at[pl.ds(0, 16)], shiftv)
    plsc.subcore_barrier()

    sv = shiftv[...]

    def p1_blk(ib, _):
        r0 = s * IR1 + ib * RB
        e0 = r0 * 128

        @pl.when(e0 < E1)
        def _():
            d1 = pltpu.async_copy(srcp.at[pl.ds(r0, RB)], srcv, sem1)
            d2 = pltpu.async_copy(dstp.at[pl.ds(r0, RB)], dstv, sem2)
            d1.wait(); d2.wait()

            def p1_sub(j, _):
                es = e0 + j * SUB

                @pl.when(es < E1)
                def _():
                    g1 = pltpu.async_copy(sh_asd.at[srcv.at[j]], svbuf, sem1)
                    g2 = pltpu.async_copy(sh_asd.at[dstv.at[j]], dvbuf, sem2)
                    g1.wait(); g2.wait()

                    @plsc.parallel_loop(0, SUB, unroll=8)
                    def _(e):
                        srow = svbuf[e, :]
                        drow = _dyngather(dvbuf[e, :], hi_idx)
                        t = srow + drow
                        t = jnp.maximum(t, 0.2 * t)
                        z = jnp.exp(t - sv)
                        padf = jnp.clip((E1 - (es + e)).astype(jnp.float32),
                                        0.0, 1.0)
                        svbuf[e, :] = z * (lo_maskf * padf)

                    w1 = pltpu.async_copy(svbuf, expalpha.at[pl.ds(es, SUB)],
                                          sem1)
                    w2 = pltpu.async_copy(svbuf, sh_den.at[dstv.at[j]],
                                          sem2, add=True)
                    w1.wait(); w2.wait()
                return 0
            lax.fori_loop(0, RB, p1_sub, 0)
        return 0
    lax.fori_loop(0, IR1 // RB, p1_blk, 0)
    plsc.subcore_barrier()

    ev, dbuf = svbuf, dvbuf

    def p2_blk(ib, _):
        r0 = w * IR2 + ib * RB
        e0 = r0 * 128

        @pl.when(e0 < E1)
        def _():
            d1 = pltpu.async_copy(srcp.at[pl.ds(r0, RB)], srcv, sem1)
            d2 = pltpu.async_copy(dstp.at[pl.ds(r0, RB)], dstv, sem2)
            d1.wait(); d2.wait()

            def p2_sub(j, _):
                es = e0 + j * SUB

                @pl.when(es < E1)
                def _():
                    g1 = pltpu.async_copy(expalpha.at[pl.ds(es, SUB)],
                                          ev, sem1)
                    g2 = pltpu.async_copy(xw.at[srcv.at[j]], xwbuf, sem2)
                    g3 = pltpu.async_copy(sh_den.at[dstv.at[j]], dbuf, sem3)
                    g1.wait(); g2.wait(); g3.wait()

                    @plsc.parallel_loop(0, SUB, unroll=4)
                    def _(e):
                        a = ev[e, :] / (dbuf[e, :] + 1e-16)
                        ev[e, :] = a
                        for h in range(H):
                            sp = jnp.full((16,), a[h], jnp.float32)
                            xwbuf[e, pl.ds(h * 16, 16)] = (
                                xwbuf[e, pl.ds(h * 16, 16)] * sp)

                    w1 = pltpu.async_copy(ev, expalpha.at[pl.ds(es, SUB)],
                                          sem1)
                    w2 = pltpu.async_copy(xwbuf, sh_out.at[dstv.at[j]],
                                          sem2, add=True)
                    w1.wait(); w2.wait()
                return 0
            lax.fori_loop(0, RB, p2_sub, 0)
        return 0
    lax.fori_loop(0, IR2 // RB, p2_blk, 0)
    plsc.subcore_barrier()

    pltpu.sync_copy(sh_out.at[pl.ds(s * NZ, NZ)],
                    outacc.at[c].at[pl.ds(s * NZ, NZ)])

    @pl.when(s == 0)
    def _():
        pltpu.sync_copy(sh_out.at[pl.ds(NS * NZ, N - NS * NZ)],
                        outacc.at[c].at[pl.ds(NS * NZ, N - NS * NZ)])


_sc_edge = functools.partial(
    pl.kernel,
    out_type=(
        jax.ShapeDtypeStruct((EP, 16), jnp.float32),
        jax.ShapeDtypeStruct((NC, N, HC), jnp.float32),
    ),
    mesh=plsc.VectorSubcoreMesh(core_axis_name="c", subcore_axis_name="s"),
    compiler_params=pltpu.CompilerParams(use_tc_tiling_on_sc=False),
    scratch_types=[
        pltpu.VMEM_SHARED((N, HC), jnp.float32),
        pltpu.VMEM_SHARED((N, 16), jnp.float32),
        pltpu.VMEM_SHARED((N, 16), jnp.float32),
        pltpu.VMEM((RB, 128), jnp.int32),
        pltpu.VMEM((RB, 128), jnp.int32),
        pltpu.VMEM((SUB, 16), jnp.float32),
        pltpu.VMEM((SUB, 16), jnp.float32),
        pltpu.VMEM((SUB, HC), jnp.float32),
        pltpu.VMEM((16,), jnp.float32),
        pltpu.SemaphoreType.DMA,
        pltpu.SemaphoreType.DMA,
        pltpu.SemaphoreType.DMA,
    ],
)(_sc_body)


def kernel(x, edge_index, W, att_src, att_dst, bias):
    f32 = jnp.float32
    loops = jnp.arange(N, dtype=jnp.int32)
    pad = jnp.zeros((EP - E1,), dtype=jnp.int32)
    srcf = jnp.concatenate([edge_index[0], loops, pad])
    dstf = jnp.concatenate([edge_index[1], loops, pad])
    srcp = srcf.reshape(EP // 128, 128)
    dstp = dstf.reshape(EP // 128, 128)

    eye = jnp.eye(H, dtype=f32)
    m_src = (att_src[:, :, None] * eye[:, None, :]).reshape(HC, H)
    m_dst = (att_dst[:, :, None] * eye[:, None, :]).reshape(HC, H)
    mcat = jnp.concatenate([m_src, m_dst], axis=1)
    p128 = jnp.tile(eye, (2, 16))

    xw, asd, srow = pl.pallas_call(
        _tc_prep,
        out_shape=(
            jax.ShapeDtypeStruct((N, HC), f32),
            jax.ShapeDtypeStruct((N, 16), f32),
            jax.ShapeDtypeStruct((1, 128), f32),
        ),
    )(x, W, mcat, p128)

    shift = srow.reshape(128)
    expalpha, outacc = _sc_edge(asd, xw, srcp, dstp, shift)

    out = pl.pallas_call(
        _tc_final,
        out_shape=jax.ShapeDtypeStruct((N, HC), f32),
    )(outacc, x, bias.reshape(1, HC))

    return out, expalpha[:E1, :8]
